# Initial kernel scaffold; baseline (speedup 1.0000x reference)
import jax
import jax.numpy as jnp
from jax.experimental import pallas as pl


def kernel(emb_weight, edge_index, edge_weight):
    raise NotImplementedError("write your pallas kernel here")


# speedup vs baseline: 2.6784x; 2.6784x over previous
import functools

import numpy as np
import jax
import jax.numpy as jnp
from jax import lax
from jax.experimental import pallas as pl
from jax.experimental.pallas import tpu as pltpu
from jax.experimental.pallas import tpu_sc as plsc

MIN_NORM = 1e-15
EPS = 1e-7
N = 10000
D = 128
H = 64
E = 320000
NC = 2
NS = 16
CH = 128
TPT = 158
EPT = CH * TPT
EP = NS * EPT
RPT = 624
TAIL = N - NS * RPT

_Z = np.int32(0)
_R = 2000
_NB = N // _R


def _tangent_body(x_ref, o_ref):
    f32 = jnp.float32
    x = x_ref[...]
    sq = jnp.sum(x * x, axis=1, keepdims=True) - x[:, 0:1] * x[:, 0:1]
    x0 = jnp.sqrt(jnp.clip(sq + f32(1.0), f32(EPS), None))
    yn = jnp.clip(jnp.sqrt(sq), f32(MIN_NORM), None)
    th = jnp.clip(x0, f32(1.0 + EPS), None)
    r = jnp.log(th + jnp.sqrt(th * th - f32(1.0))) / yn
    h = pl.program_id(0)
    col = lax.broadcasted_iota(jnp.int32, (_R, D), 1)
    full = jnp.where(col == 0, f32(0.0), x * r)
    half = lax.cond(h == 0, lambda: full[:, :H], lambda: full[:, H:])
    o_ref[...] = half


def _tangent(emb):
    return pl.pallas_call(
        _tangent_body,
        grid=(2, _NB),
        in_specs=[
            pl.BlockSpec((_R, D), lambda h, i: (i, _Z)),
        ],
        out_specs=pl.BlockSpec((_R, H), lambda h, i: (h * _NB + i, _Z)),
        out_shape=jax.ShapeDtypeStruct((NC * N, H), jnp.float32),
    )(emb)


@functools.lru_cache(maxsize=None)
def _make_spmm():
    mesh = plsc.VectorSubcoreMesh(core_axis_name="c", subcore_axis_name="s")
    return functools.partial(
        pl.kernel,
        out_type=jax.ShapeDtypeStruct((NC * N, H), jnp.float32),
        mesh=mesh,
        scratch_types=[
            pltpu.VMEM((TPT, CH), jnp.int32),
            pltpu.VMEM((TPT, CH), jnp.int32),
            pltpu.VMEM((TPT, CH), jnp.float32),
            pltpu.VMEM((CH, H), jnp.float32),
            pltpu.VMEM_SHARED((N, H), jnp.float32),
            pltpu.SemaphoreType.DMA,
        ],
        compiler_params=pltpu.CompilerParams(use_tc_tiling_on_sc=False),
    )(_spmm_body)


def _spmm_body(table, srcs, dsts, ws, zblk, out, src_v, dst_v, w_v, rows, acc, sem):
    i32 = jnp.int32
    cid = lax.axis_index("c").astype(i32)
    sid = lax.axis_index("s").astype(i32)
    arow = sid * i32(RPT)
    pltpu.sync_copy(zblk, acc.at[pl.ds(arow, RPT)])

    @pl.when(sid == 0)
    def _zero_tail():
        pltpu.sync_copy(zblk.at[pl.ds(0, TAIL)], acc.at[pl.ds(NS * RPT, TAIL)])
    pltpu.sync_copy(srcs.at[cid, sid], src_v)
    pltpu.sync_copy(dsts.at[sid], dst_v)
    pltpu.sync_copy(ws.at[sid], w_v)
    plsc.subcore_barrier()

    def chunk(t, carry):
        pltpu.async_copy(table.at[src_v.at[t]], rows, sem).wait()

        def grp(g, c2):
            gbase = g * i32(16)
            wrow = w_v[t, pl.ds(gbase, 16)]
            for j in range(16):
                wv = wrow[j]
                e = gbase + i32(j)
                for q in range(4):
                    rows[e, pl.ds(q * 16, 16)] = rows[e, pl.ds(q * 16, 16)] * wv
            return c2

        lax.fori_loop(i32(0), i32(CH // 16), grp, i32(0))
        pltpu.sync_copy(rows, acc.at[dst_v.at[t]], add=True)
        return carry

    lax.fori_loop(i32(0), i32(TPT), chunk, i32(0))
    plsc.subcore_barrier()
    pltpu.sync_copy(acc.at[pl.ds(arow, RPT)],
                    out.at[pl.ds(cid * i32(N) + arow, RPT)])

    @pl.when(sid == 0)
    def _copy_tail():
        pltpu.sync_copy(acc.at[pl.ds(NS * RPT, TAIL)],
                        out.at[pl.ds(cid * i32(N) + i32(NS * RPT), TAIL)])


def _decode_body(a0, b0, a1, b1, o_ref):
    f32 = jnp.float32
    y0 = a0[...] + b0[...]
    y1 = a1[...] + b1[...]
    y = jnp.concatenate([y0, y1], axis=1)
    col = lax.broadcasted_iota(jnp.int32, (_R, D), 1)
    u = jnp.where(col == 0, f32(0.0), y)
    un = jnp.clip(jnp.sqrt(jnp.sum(u * u, axis=1, keepdims=True)),
                  f32(MIN_NORM), None)
    s = (jnp.exp(un) - jnp.exp(-un)) * f32(0.5) / un
    res = s * u
    x0 = jnp.sqrt(jnp.clip(jnp.sum(res * res, axis=1, keepdims=True) + f32(1.0),
                           f32(EPS), None))
    o_ref[...] = jnp.where(col == 0, x0, res)


def _decode(t1, t2):
    return pl.pallas_call(
        _decode_body,
        grid=(_NB,),
        in_specs=[
            pl.BlockSpec((_R, H), lambda i: (i, _Z)),
            pl.BlockSpec((_R, H), lambda i: (i, _Z)),
            pl.BlockSpec((_R, H), lambda i: (_NB + i, _Z)),
            pl.BlockSpec((_R, H), lambda i: (_NB + i, _Z)),
        ],
        out_specs=pl.BlockSpec((_R, D), lambda i: (i, _Z)),
        out_shape=jax.ShapeDtypeStruct((N, D), jnp.float32),
    )(t1, t2, t1, t2)


def kernel(emb_weight, edge_index, edge_weight):
    emb = emb_weight.astype(jnp.float32)
    src = edge_index[1].astype(jnp.int32)
    dst = edge_index[0].astype(jnp.int32)
    w = edge_weight.astype(jnp.float32)
    pad = EP - E
    src = jnp.pad(src, (0, pad)).reshape(NS, TPT, CH)
    dst = jnp.pad(dst, (0, pad)).reshape(NS, TPT, CH)
    w = jnp.pad(w, (0, pad)).reshape(NS, TPT, CH)
    srcs = jnp.stack([src, src + N])
    zblk = jnp.zeros((RPT, H), jnp.float32)

    spmm = _make_spmm()
    xt = _tangent(emb)
    t1 = spmm(xt, srcs, dst, w, zblk)
    t2 = spmm(t1, srcs, dst, w, zblk)
    return _decode(t1, t2)
